# Initial kernel scaffold; baseline (speedup 1.0000x reference)
import jax
import jax.numpy as jnp
from jax.experimental import pallas as pl


def kernel(risk, time, event):
    raise NotImplementedError("write your pallas kernel here")


# speedup vs baseline: 1.1011x; 1.1011x over previous
import jax
import jax.numpy as jnp
from jax.experimental import pallas as pl
from jax.experimental.pallas import tpu as pltpu

_N = 16384
_BI = 256
_JC = 2048
_NBLK = _N // _BI


def _cox_block(t_col_ref, t_row_ref, r_row_ref, r_col_ref, e_col_ref,
               num_ref, den_ref):
    ti = t_col_ref[...]
    expj = jnp.exp(r_row_ref[...])
    trow = t_row_ref[...]
    acc = jnp.zeros((_BI, _JC), jnp.float32)
    for c in range(_N // _JC):
        tj = trow[:, c * _JC:(c + 1) * _JC]
        ej = expj[:, c * _JC:(c + 1) * _JC]
        acc = acc + jnp.where(tj >= ti, ej, 0.0)
    s = acc.sum(axis=1, keepdims=True)
    contrib = (r_col_ref[...] - jnp.log(s)) * e_col_ref[...]
    num_ref[...] = contrib.sum(keepdims=True).reshape(1, 1, 1)
    den_ref[...] = e_col_ref[...].sum(keepdims=True).reshape(1, 1, 1)


def kernel(risk, time, event):
    t_row = time.reshape(1, _N)
    r_row = risk.reshape(1, _N)
    t_col = time.reshape(_N, 1)
    r_col = risk.reshape(_N, 1)
    e_col = event.reshape(_N, 1)

    num, den = pl.pallas_call(
        _cox_block,
        grid=(_NBLK,),
        in_specs=[
            pl.BlockSpec((_BI, 1), lambda b: (b, 0)),
            pl.BlockSpec((1, _N), lambda b: (0, 0)),
            pl.BlockSpec((1, _N), lambda b: (0, 0)),
            pl.BlockSpec((_BI, 1), lambda b: (b, 0)),
            pl.BlockSpec((_BI, 1), lambda b: (b, 0)),
        ],
        out_specs=[
            pl.BlockSpec((1, 1, 1), lambda b: (b, 0, 0)),
            pl.BlockSpec((1, 1, 1), lambda b: (b, 0, 0)),
        ],
        out_shape=[
            jax.ShapeDtypeStruct((_NBLK, 1, 1), jnp.float32),
            jax.ShapeDtypeStruct((_NBLK, 1, 1), jnp.float32),
        ],
        compiler_params=pltpu.CompilerParams(
            dimension_semantics=("parallel",),
        ),
    )(t_col, t_row, r_row, r_col, e_col)

    return -(num.sum() / den.sum())


# speedup vs baseline: 1.1195x; 1.0168x over previous
import jax
import jax.numpy as jnp
from jax.experimental import pallas as pl
from jax.experimental.pallas import tpu as pltpu

_N = 16384
_BI = 256
_JC = 2048
_NBLK = _N // _BI


def _cox_block(t_col_ref, t_row_ref, r_row_ref, r_col_ref, e_col_ref,
               num_ref, den_ref):
    ti = t_col_ref[...]
    expj = jnp.exp(r_row_ref[...])
    trow = t_row_ref[...]
    s = jnp.zeros((_BI, 1), jnp.float32)
    for c in range(_N // _JC):
        tj = trow[:, c * _JC:(c + 1) * _JC]
        ej = expj[:, c * _JC:(c + 1) * _JC]
        s = s + jnp.where(tj >= ti, ej, 0.0).sum(axis=1, keepdims=True)
    contrib = (r_col_ref[...] - jnp.log(s)) * e_col_ref[...]
    num_ref[...] = contrib.sum(keepdims=True).reshape(1, 1, 1)
    den_ref[...] = e_col_ref[...].sum(keepdims=True).reshape(1, 1, 1)


def kernel(risk, time, event):
    t_row = time.reshape(1, _N)
    r_row = risk.reshape(1, _N)
    t_col = time.reshape(_N, 1)
    r_col = risk.reshape(_N, 1)
    e_col = event.reshape(_N, 1)

    num, den = pl.pallas_call(
        _cox_block,
        grid=(_NBLK,),
        in_specs=[
            pl.BlockSpec((_BI, 1), lambda b: (b, 0)),
            pl.BlockSpec((1, _N), lambda b: (0, 0)),
            pl.BlockSpec((1, _N), lambda b: (0, 0)),
            pl.BlockSpec((_BI, 1), lambda b: (b, 0)),
            pl.BlockSpec((_BI, 1), lambda b: (b, 0)),
        ],
        out_specs=[
            pl.BlockSpec((1, 1, 1), lambda b: (b, 0, 0)),
            pl.BlockSpec((1, 1, 1), lambda b: (b, 0, 0)),
        ],
        out_shape=[
            jax.ShapeDtypeStruct((_NBLK, 1, 1), jnp.float32),
            jax.ShapeDtypeStruct((_NBLK, 1, 1), jnp.float32),
        ],
        compiler_params=pltpu.CompilerParams(
            dimension_semantics=("arbitrary",),
        ),
    )(t_col, t_row, r_row, r_col, e_col)

    return -(num.sum() / den.sum())


# speedup vs baseline: 1.9762x; 1.7653x over previous
import jax
import jax.numpy as jnp
from jax.experimental import pallas as pl
from jax.experimental.pallas import tpu as pltpu

_N = 16384
_BI = 4096
_JC = 1024
_NBLK = _N // _BI


def _cox_block(t_col_ref, t_blk_ref, r_row_ref, r_blk_ref, e_blk_ref,
               num_ref, den_ref):
    ti = t_blk_ref[...]
    expj = jnp.exp(r_row_ref[...])
    eh = expj.astype(jnp.bfloat16)
    el = (expj - eh.astype(jnp.float32)).astype(jnp.bfloat16)
    ehl = jnp.concatenate([eh, el], axis=0)

    s2 = jnp.zeros((2, _BI), jnp.float32)
    for c in range(_N // _JC):
        tj = t_col_ref[c * _JC:(c + 1) * _JC, :]
        mask = jnp.where(tj >= ti, 1.0, 0.0).astype(jnp.bfloat16)
        lhs = ehl[:, c * _JC:(c + 1) * _JC]
        s2 = s2 + jax.lax.dot_general(
            lhs, mask, (((1,), (0,)), ((), ())),
            preferred_element_type=jnp.float32)
    s = s2[0:1, :] + s2[1:2, :]
    contrib = (r_blk_ref[...] - jnp.log(s)) * e_blk_ref[...]
    num_ref[...] = contrib.sum(keepdims=True).reshape(1, 1, 1)
    den_ref[...] = e_blk_ref[...].sum(keepdims=True).reshape(1, 1, 1)


def kernel(risk, time, event):
    t_col = time.reshape(_N, 1)
    t_row = time.reshape(1, _N)
    r_row = risk.reshape(1, _N)
    e_row = event.reshape(1, _N)

    num, den = pl.pallas_call(
        _cox_block,
        grid=(_NBLK,),
        in_specs=[
            pl.BlockSpec((_N, 1), lambda b: (0, 0)),
            pl.BlockSpec((1, _BI), lambda b: (0, b)),
            pl.BlockSpec((1, _N), lambda b: (0, 0)),
            pl.BlockSpec((1, _BI), lambda b: (0, b)),
            pl.BlockSpec((1, _BI), lambda b: (0, b)),
        ],
        out_specs=[
            pl.BlockSpec((1, 1, 1), lambda b: (b, 0, 0)),
            pl.BlockSpec((1, 1, 1), lambda b: (b, 0, 0)),
        ],
        out_shape=[
            jax.ShapeDtypeStruct((_NBLK, 1, 1), jnp.float32),
            jax.ShapeDtypeStruct((_NBLK, 1, 1), jnp.float32),
        ],
        compiler_params=pltpu.CompilerParams(
            dimension_semantics=("arbitrary",),
        ),
    )(t_col, t_row, r_row, r_row, e_row)

    return -(num.sum() / den.sum())
